# baseline (speedup 1.0000x reference)
import jax
import jax.numpy as jnp
from jax.experimental import pallas as pl

G = 64
MIN_SCORE = 0.001


def _mm(x, W, BM=1000):
    M, K = x.shape
    _, NO = W.shape

    def body(xr, wr, outr):
        outr[...] = jnp.dot(xr[...], wr[...],
                            preferred_element_type=jnp.float32)

    return pl.pallas_call(
        body,
        grid=(M // BM,),
        in_specs=[
            pl.BlockSpec((BM, K), lambda i: (i, 0)),
            pl.BlockSpec((K, NO), lambda i: (0, 0)),
        ],
        out_specs=pl.BlockSpec((BM, NO), lambda i: (i, 0)),
        out_shape=jax.ShapeDtypeStruct((M, NO), jnp.float32),
    )(x, W)


def _gcn(x, src, dst, W, b):
    n = x.shape[0]
    h = _mm(x, W)
    loop = jnp.arange(n, dtype=src.dtype)
    s = jnp.concatenate([src, loop])
    d = jnp.concatenate([dst, loop])
    deg = jnp.zeros((n,), h.dtype).at[d].add(1.0)
    dinv = jnp.where(deg > 0, deg ** -0.5, 0.0)
    norm = dinv[s] * dinv[d]
    out = jnp.zeros_like(h).at[d].add(h[s] * norm[:, None])
    return out + b


def _gcn_masked(x, src, dst, W, b, mask):
    n = x.shape[0]
    h = _mm(x, W)
    mf = mask.astype(h.dtype)
    ew = mf[src] * mf[dst]
    deg = jnp.zeros((n,), h.dtype).at[dst].add(ew) + mf
    dinv = jnp.where(deg > 0, deg ** -0.5, 0.0)
    norm = dinv[src] * dinv[dst] * ew
    out = jnp.zeros_like(h).at[dst].add(h[src] * norm[:, None])
    out = out + h * ((dinv * dinv) * mf)[:, None]
    return out + b


def _gcnnet(x, edge_index, batch, p):
    src, dst = edge_index[0], edge_index[1]
    h = jax.nn.elu(_gcn(x, src, dst, p['W1'], p['b1']))
    h = jax.nn.elu(_gcn(h, src, dst, p['W2'], p['b2']))
    Wp_pad = jnp.pad(p['Wp'], ((0, 0), (0, 127)))
    s = _gcn(h, src, dst, Wp_pad, jnp.pad(p['bp'], (0, 127)))[:, 0]
    smax = jax.ops.segment_max(s, batch, num_segments=G)
    e = jnp.exp(s - smax[batch])
    z = jax.ops.segment_sum(e, batch, num_segments=G)
    score = e / (z[batch] + 1e-16)
    scmax = jax.ops.segment_max(score, batch, num_segments=G)
    thresh = jnp.minimum(scmax[batch] - 1e-7, MIN_SCORE)
    mask = score > thresh
    mf = mask.astype(h.dtype)
    hx = h * (score * mf)[:, None]
    h3 = _gcn_masked(hx, src, dst, p['W3'], p['b3'], mask)
    h3 = jax.nn.elu(h3) * mf[:, None]
    return jax.ops.segment_sum(h3, batch, num_segments=G)


def _head(x1, x2, Wc1, bc1, Wc2, bc2, Wc3, bc3):
    def body(xx_r, w1r, b1r, w2r, b2r, w3r, b3r, outr):
        hh = jnp.maximum(jnp.dot(xx_r[...], w1r[...],
                                 preferred_element_type=jnp.float32)
                         + b1r[...], 0.0)
        hh = jnp.maximum(jnp.dot(hh, w2r[...],
                                 preferred_element_type=jnp.float32)
                         + b2r[...], 0.0)
        outr[...] = (jnp.dot(hh, w3r[...],
                             preferred_element_type=jnp.float32)
                     + b3r[...])

    xx = jnp.concatenate([x1, x2, x1 - x2, x1 * x2], axis=-1)
    Wc3p = jnp.pad(Wc3, ((0, 0), (0, 127)))
    bc3p = jnp.pad(bc3.reshape(1, 1), ((0, 0), (0, 127)))
    out = pl.pallas_call(
        body,
        out_shape=jax.ShapeDtypeStruct((G, 128), jnp.float32),
    )(xx, Wc1, bc1.reshape(1, 64), Wc2, bc2.reshape(1, 64), Wc3p, bc3p)
    return out[:, 0]


def kernel(x_a, edge_index_a, batch_a, x_b, edge_index_b, batch_b,
           W1, b1, W2, b2, W3, b3, Wp, bp,
           Wc1, bc1, Wc2, bc2, Wc3, bc3):
    p = {'W1': W1, 'b1': b1, 'W2': W2, 'b2': b2, 'W3': W3, 'b3': b3,
         'Wp': Wp, 'bp': bp}
    x1 = _gcnnet(x_a, edge_index_a, batch_a, p)
    x2 = _gcnnet(x_b, edge_index_b, batch_b, p)
    return _head(x1, x2, Wc1, bc1, Wc2, bc2, Wc3, bc3)


# speedup vs baseline: 14.3815x; 14.3815x over previous
import functools

import jax
import jax.numpy as jnp
from jax import lax
from jax.experimental import pallas as pl
from jax.experimental.pallas import tpu as pltpu
from jax.experimental.pallas import tpu_sc as plsc

N = 10000
NP = 10240
E = 160000
EP = 163840
G = 64
MIN_SCORE = 0.001
NC = 2
NS = 16
ERT = EP // 128
CPT4 = EP // (NS * 128)
CPT1 = EP // (NC * NS * 128)
RPT = NP // NS
EPT_S = EP // (NC * NS)
MB = NP // 1024
BM = 1024

_f32 = jnp.float32
_i32 = jnp.int32


def _sc_mesh():
    return plsc.VectorSubcoreMesh(core_axis_name="c", subcore_axis_name="s",
                                  num_cores=NC, num_subcores=NS)


def _scatter_wide4(u, srcg, dst, z):
    def body(u_hbm, srcg_hbm, dst_hbm, z_hbm, out_hbm,
             sidx, didx, buf0, buf1, acc, sem0, sem1):
        c = lax.axis_index("c")
        s = lax.axis_index("s")
        HC = CPT4 // 2
        pltpu.sync_copy(dst_hbm.at[pl.ds(s * CPT4, CPT4)], didx)
        for r in range(4 // NC):
            cb = NC * r + c
            pltpu.sync_copy(z_hbm.at[pl.ds(s * RPT, RPT)],
                            acc.at[pl.ds(s * RPT, RPT)])
            plsc.subcore_barrier()
            for half in range(2):
                dbase = half * HC
                pltpu.sync_copy(
                    srcg_hbm.at[pl.ds(cb * ERT + s * CPT4 + dbase, HC)],
                    sidx)
                pltpu.async_copy(u_hbm.at[sidx.at[0]], buf0, sem0)

                def step(g, carry, dbase=dbase):
                    j0 = 2 * g
                    pltpu.async_copy(u_hbm.at[sidx.at[j0 + 1]], buf1, sem1)
                    pltpu.make_async_copy(u_hbm.at[sidx.at[j0]], buf0,
                                          sem0).wait()
                    pltpu.sync_copy(buf0, acc.at[didx.at[dbase + j0]],
                                    add=True)

                    @pl.when(j0 + 2 < HC)
                    def _():
                        pltpu.async_copy(u_hbm.at[sidx.at[j0 + 2]], buf0,
                                         sem0)

                    pltpu.make_async_copy(u_hbm.at[sidx.at[j0 + 1]], buf1,
                                          sem1).wait()
                    pltpu.sync_copy(buf1, acc.at[didx.at[dbase + j0 + 1]],
                                    add=True)
                    return carry

                lax.fori_loop(0, HC // 2, step, 0)
            plsc.subcore_barrier()
            pltpu.sync_copy(acc.at[pl.ds(s * RPT, RPT)],
                            out_hbm.at[pl.ds(cb * NP + s * RPT, RPT)])
            plsc.subcore_barrier()

    fn = pl.kernel(
        body,
        out_type=jax.ShapeDtypeStruct((4 * NP, 128), _f32),
        mesh=_sc_mesh(),
        scratch_types=[
            pltpu.VMEM((CPT4 // 2, 128), _i32),
            pltpu.VMEM((CPT4, 128), _i32),
            pltpu.VMEM((128, 128), _f32),
            pltpu.VMEM((128, 128), _f32),
            pltpu.VMEM_SHARED((NP, 128), _f32),
            pltpu.SemaphoreType.DMA,
            pltpu.SemaphoreType.DMA,
        ],
    )
    return fn(u, srcg, dst, z)


def _scatter_wide1(u, src, dst, z, W=128):
    def body(u_hbm, src_hbm, dst_hbm, z_hbm, out_hbm,
             sidx, didx, buf0, buf1, acc, sem0, sem1):
        c = lax.axis_index("c")
        s = lax.axis_index("s")
        base = (c * NS + s) * CPT1
        pltpu.sync_copy(src_hbm.at[pl.ds(base, CPT1)], sidx)
        pltpu.sync_copy(dst_hbm.at[pl.ds(base, CPT1)], didx)
        pltpu.sync_copy(z_hbm.at[pl.ds(s * RPT, RPT)],
                        acc.at[pl.ds(s * RPT, RPT)])
        plsc.subcore_barrier()
        pltpu.async_copy(u_hbm.at[sidx.at[0]], buf0, sem0)

        def step(g, carry):
            j0 = 2 * g
            pltpu.async_copy(u_hbm.at[sidx.at[j0 + 1]], buf1, sem1)
            pltpu.make_async_copy(u_hbm.at[sidx.at[j0]], buf0, sem0).wait()
            pltpu.sync_copy(buf0, acc.at[didx.at[j0]], add=True)

            @pl.when(j0 + 2 < CPT1)
            def _():
                pltpu.async_copy(u_hbm.at[sidx.at[j0 + 2]], buf0, sem0)

            pltpu.make_async_copy(u_hbm.at[sidx.at[j0 + 1]], buf1,
                                  sem1).wait()
            pltpu.sync_copy(buf1, acc.at[didx.at[j0 + 1]], add=True)
            return carry

        lax.fori_loop(0, CPT1 // 2, step, 0)
        plsc.subcore_barrier()
        pltpu.sync_copy(acc.at[pl.ds(s * RPT, RPT)],
                        out_hbm.at[pl.ds(c * NP + s * RPT, RPT)])

    fn = pl.kernel(
        body,
        out_type=jax.ShapeDtypeStruct((NC * NP, W), _f32),
        mesh=_sc_mesh(),
        scratch_types=[
            pltpu.VMEM((CPT1, 128), _i32),
            pltpu.VMEM((CPT1, 128), _i32),
            pltpu.VMEM((128, W), _f32),
            pltpu.VMEM((128, W), _f32),
            pltpu.VMEM_SHARED((NP, W), _f32),
            pltpu.SemaphoreType.DMA,
            pltpu.SemaphoreType.DMA,
        ],
    )
    return fn(u, src, dst, z)


def _scatter_col0(u, src2, dst2, z):
    return _scatter_wide1(u, src2, dst2, z, W=128)


def _elu(x):
    return jnp.where(x > 0, x, jnp.exp(x) - 1.0)


def _tk1(x, W1, degp):
    F = x.shape[1]

    def body(x_r, w_r, dp_r, u1_r, h0_r, dinv_r):
        h0b = jnp.dot(x_r[...], w_r[...], preferred_element_type=_f32)
        dp = dp_r[...]
        deg = (dp[0] + dp[1])[:, :1] + 1.0
        y = lax.rsqrt(deg)
        dinvb = y * (1.5 - 0.5 * deg * y * y)
        u1_r[...] = dinvb * h0b
        h0_r[...] = h0b
        dinv_r[...] = dinvb

    return pl.pallas_call(
        body,
        grid=(MB, 4),
        in_specs=[
            pl.BlockSpec((BM, F), lambda i, j: (i, 0)),
            pl.BlockSpec((F, 128), lambda i, j: (0, j)),
            pl.BlockSpec((2, BM, 128), lambda i, j: (0, i, 0)),
        ],
        out_specs=[
            pl.BlockSpec((BM, 128), lambda i, j: (j * MB + i, 0)),
            pl.BlockSpec((BM, 128), lambda i, j: (i, j)),
            pl.BlockSpec((BM, 1), lambda i, j: (i, 0)),
        ],
        out_shape=[
            jax.ShapeDtypeStruct((4 * NP, 128), _f32),
            jax.ShapeDtypeStruct((NP, 512), _f32),
            jax.ShapeDtypeStruct((NP, 1), _f32),
        ],
    )(x, W1, degp)


def _tk_layer(v, hwprev, dinv, W, b):
    def body(v_r, hp_r, dinv_r, w_r, b_r, u_r, hw_r):
        vv = v_r[...]
        cat = jnp.concatenate([vv[0], vv[1], vv[2], vv[3]], axis=-1)
        d = dinv_r[...]
        h = _elu(d * cat + d * d * hp_r[...] + b_r[...])
        hwb = jnp.dot(h, w_r[...], preferred_element_type=_f32)
        u_r[...] = d * hwb
        hw_r[...] = hwb

    return pl.pallas_call(
        body,
        grid=(MB, 4),
        in_specs=[
            pl.BlockSpec((4, BM, 128), lambda i, j: (0, i, 0)),
            pl.BlockSpec((BM, 512), lambda i, j: (i, 0)),
            pl.BlockSpec((BM, 1), lambda i, j: (i, 0)),
            pl.BlockSpec((512, 128), lambda i, j: (0, j)),
            pl.BlockSpec((1, 512), lambda i, j: (0, 0)),
        ],
        out_specs=[
            pl.BlockSpec((BM, 128), lambda i, j: (j * MB + i, 0)),
            pl.BlockSpec((BM, 128), lambda i, j: (i, j)),
        ],
        out_shape=[
            jax.ShapeDtypeStruct((4 * NP, 128), _f32),
            jax.ShapeDtypeStruct((NP, 512), _f32),
        ],
    )(v.reshape(4, NP, 128), hwprev, dinv, W, b.reshape(1, 512))


def _tk3a(v2, h1, dinv, b2, Wpp):
    def body(v_r, hp_r, dinv_r, b_r, wp_r, h2_r, t0_r, ut_r):
        vv = v_r[...]
        cat = jnp.concatenate([vv[0], vv[1], vv[2], vv[3]], axis=-1)
        d = dinv_r[...]
        h2 = _elu(d * cat + d * d * hp_r[...] + b_r[...])
        tp = jnp.dot(h2, wp_r[...], preferred_element_type=_f32)
        t0 = tp[:, :1]
        h2_r[...] = h2
        t0_r[...] = t0
        col0 = lax.broadcasted_iota(_i32, (BM, 128), 1) == 0
        ut_r[...] = jnp.where(col0, d * t0, 0.0)

    return pl.pallas_call(
        body,
        grid=(MB,),
        in_specs=[
            pl.BlockSpec((4, BM, 128), lambda i: (0, i, 0)),
            pl.BlockSpec((BM, 512), lambda i: (i, 0)),
            pl.BlockSpec((BM, 1), lambda i: (i, 0)),
            pl.BlockSpec((1, 512), lambda i: (0, 0)),
            pl.BlockSpec((512, 128), lambda i: (0, 0)),
        ],
        out_specs=[
            pl.BlockSpec((BM, 512), lambda i: (i, 0)),
            pl.BlockSpec((BM, 1), lambda i: (i, 0)),
            pl.BlockSpec((BM, 128), lambda i: (i, 0)),
        ],
        out_shape=[
            jax.ShapeDtypeStruct((NP, 512), _f32),
            jax.ShapeDtypeStruct((NP, 1), _f32),
            jax.ShapeDtypeStruct((NP, 128), _f32),
        ],
    )(v2.reshape(4, NP, 128), h1, dinv, b2.reshape(1, 512), Wpp)


def _tk_col0sum(vp):
    def body(vp_r, out_r):
        v = vp_r[...]
        out_r[...] = (v[0] + v[1])[:, :1]

    return pl.pallas_call(
        body,
        grid=(MB,),
        in_specs=[pl.BlockSpec((2, BM, 128), lambda i: (0, i, 0))],
        out_specs=pl.BlockSpec((BM, 1), lambda i: (i, 0)),
        out_shape=jax.ShapeDtypeStruct((NP, 1), _f32),
    )(vp)


def _tk3b(vs0, t0, dinv, batch2, bp):
    def body(vs_r, t0_r, dinv_r, b_r, bp_r, mf_r, sm_r, mf16_r):
        vs = vs_r[...]
        d = dinv_r[...]
        s = d * vs + d * d * t0_r[...] + bp_r[0, 0]
        bat = b_r[...]
        iota_g = lax.broadcasted_iota(_i32, (1, G), 1)
        B = (bat == iota_g).astype(_f32)
        valid = bat < G
        neg = jnp.float32(-1e30)
        smax_g = jnp.max(jnp.where(B > 0, s, neg), axis=0, keepdims=True)
        smax_n = lax.dot_general(B, smax_g, (((1,), (1,)), ((), ())),
                                 preferred_element_type=_f32)
        e = jnp.where(valid, jnp.exp(s - smax_n), 0.0)
        z_g = lax.dot_general(B, e, (((0,), (0,)), ((), ())),
                              preferred_element_type=_f32)
        z_n = lax.dot_general(B, z_g, (((1,), (0,)), ((), ())),
                              preferred_element_type=_f32)
        score = e / (z_n + 1e-16)
        scmax_g = jnp.max(jnp.where(B > 0, score, neg), axis=0,
                          keepdims=True)
        scmax_n = lax.dot_general(B, scmax_g, (((1,), (1,)), ((), ())),
                                  preferred_element_type=_f32)
        thresh = jnp.minimum(scmax_n - 1e-7, MIN_SCORE)
        sel = jnp.logical_and(valid, score > thresh)
        mfv = jnp.where(sel, 1.0, 0.0)
        mf_r[...] = mfv
        sm_r[...] = jnp.where(sel, score, 0.0)
        col0 = lax.broadcasted_iota(_i32, (NP, 128), 1) == 0
        mf16_r[...] = jnp.where(col0, mfv, 0.0)

    return pl.pallas_call(
        body,
        in_specs=[
            pl.BlockSpec((NP, 1), lambda: (0, 0)),
            pl.BlockSpec((NP, 1), lambda: (0, 0)),
            pl.BlockSpec((NP, 1), lambda: (0, 0)),
            pl.BlockSpec((NP, 1), lambda: (0, 0)),
            pl.BlockSpec((1, 1), lambda: (0, 0)),
        ],
        out_specs=[
            pl.BlockSpec((NP, 1), lambda: (0, 0)),
            pl.BlockSpec((NP, 1), lambda: (0, 0)),
            pl.BlockSpec((NP, 128), lambda: (0, 0)),
        ],
        out_shape=[
            jax.ShapeDtypeStruct((NP, 1), _f32),
            jax.ShapeDtypeStruct((NP, 1), _f32),
            jax.ShapeDtypeStruct((NP, 128), _f32),
        ],
    )(vs0, t0, dinv, batch2, bp)


def _tk3c(h2, sm, mf, mpart, W3):
    def body(h2_r, sm_r, mf_r, mp_r, w_r, u3_r, hw_r, d3_r):
        mp = mp_r[...]
        msum = (mp[0] + mp[1])[:, :1]
        mfv = mf_r[...]
        deg3 = mfv * msum + mfv
        y3 = lax.rsqrt(jnp.maximum(deg3, 1e-30))
        y3 = y3 * (1.5 - 0.5 * deg3 * y3 * y3)
        dinv3 = jnp.where(deg3 > 0, y3, 0.0)
        hx = h2_r[...] * sm_r[...]
        hw = jnp.dot(hx, w_r[...], preferred_element_type=_f32)
        u3_r[...] = dinv3 * hw
        hw_r[...] = hw
        d3_r[...] = dinv3

    return pl.pallas_call(
        body,
        grid=(MB,),
        in_specs=[
            pl.BlockSpec((BM, 512), lambda i: (i, 0)),
            pl.BlockSpec((BM, 1), lambda i: (i, 0)),
            pl.BlockSpec((BM, 1), lambda i: (i, 0)),
            pl.BlockSpec((2, BM, 128), lambda i: (0, i, 0)),
            pl.BlockSpec((512, 128), lambda i: (0, 0)),
        ],
        out_specs=[
            pl.BlockSpec((BM, 128), lambda i: (i, 0)),
            pl.BlockSpec((BM, 128), lambda i: (i, 0)),
            pl.BlockSpec((BM, 1), lambda i: (i, 0)),
        ],
        out_shape=[
            jax.ShapeDtypeStruct((NP, 128), _f32),
            jax.ShapeDtypeStruct((NP, 128), _f32),
            jax.ShapeDtypeStruct((NP, 1), _f32),
        ],
    )(h2, sm, mf, mpart, W3)


def _tk3d(v3p, hw, dinv3, mf, b3, batch2):
    def body(v_r, hw_r, d3_r, mf_r, b_r, bat_r, out_r):
        i = pl.program_id(0)
        vv = v_r[...]
        v3 = vv[0] + vv[1]
        d3 = d3_r[...]
        out3 = d3 * v3 + d3 * d3 * hw_r[...] + b_r[...]
        h3f = _elu(out3) * mf_r[...]
        iota_g = lax.broadcasted_iota(_i32, (1, G), 1)
        B = (bat_r[...] == iota_g).astype(_f32)
        part = lax.dot_general(B, h3f, (((0,), (0,)), ((), ())),
                               preferred_element_type=_f32)

        @pl.when(i == 0)
        def _():
            out_r[...] = part

        @pl.when(i > 0)
        def _():
            out_r[...] = out_r[...] + part

    return pl.pallas_call(
        body,
        grid=(MB,),
        in_specs=[
            pl.BlockSpec((2, BM, 128), lambda i: (0, i, 0)),
            pl.BlockSpec((BM, 128), lambda i: (i, 0)),
            pl.BlockSpec((BM, 1), lambda i: (i, 0)),
            pl.BlockSpec((BM, 1), lambda i: (i, 0)),
            pl.BlockSpec((1, 128), lambda i: (0, 0)),
            pl.BlockSpec((BM, 1), lambda i: (i, 0)),
        ],
        out_specs=pl.BlockSpec((G, 128), lambda i: (0, 0)),
        out_shape=jax.ShapeDtypeStruct((G, 128), _f32),
    )(v3p.reshape(2, NP, 128), hw, dinv3, mf, b3.reshape(1, 128), batch2)


def _head(x1, x2, Wc1, bc1, Wc2, bc2, Wc3p, bc3p):
    def body(x1_r, x2_r, w1r, b1r, w2r, b2r, w3r, b3r, outr):
        a, b = x1_r[...], x2_r[...]
        xx = jnp.concatenate([a, b, a - b, a * b], axis=-1)
        hh = jnp.maximum(jnp.dot(xx, w1r[...],
                                 preferred_element_type=_f32) + b1r[...],
                         0.0)
        hh = jnp.maximum(jnp.dot(hh, w2r[...],
                                 preferred_element_type=_f32) + b2r[...],
                         0.0)
        outr[...] = (jnp.dot(hh, w3r[...], preferred_element_type=_f32)
                     + b3r[...])

    return pl.pallas_call(
        body,
        out_shape=jax.ShapeDtypeStruct((G, 128), _f32),
    )(x1, x2, Wc1, bc1.reshape(1, 64), Wc2, bc2.reshape(1, 64),
      Wc3p, bc3p)


def _gcnnet(x, edge_index, batch, p, z):
    src = edge_index[0]
    dst = edge_index[1]
    padi = jnp.arange(EP - E, dtype=_i32)
    srcp = jnp.concatenate([src, padi % 16])
    dstp = jnp.concatenate([dst, N + (padi % (NP - N))])
    src2 = srcp.reshape(ERT, 128)
    dst2 = dstp.reshape(ERT, 128)
    srcg = jnp.concatenate(
        [srcp + cb * NP for cb in range(4)]).reshape(4 * ERT, 128)
    xp = jnp.concatenate(
        [x, jnp.zeros((NP - N, x.shape[1]), _f32)], axis=0)
    batchp = jnp.concatenate(
        [batch.astype(_i32), jnp.full((NP - N,), G, _i32)])
    batch2 = batchp[:, None]

    ones0 = jnp.pad(jnp.ones((NP, 1), _f32), ((0, 0), (0, 127)))

    degp = _scatter_col0(ones0, src2, dst2, z).reshape(2, NP, 128)
    u1, h0, dinv = _tk1(xp, p['W1'], degp)
    v1 = _scatter_wide4(u1, srcg, dst2, z)
    u2, hw2 = _tk_layer(v1, h0, dinv, p['W2'], p['b1'])
    v2 = _scatter_wide4(u2, srcg, dst2, z)
    Wpp = jnp.pad(p['Wp'], ((0, 0), (0, 127)))
    h2, t0, ut16 = _tk3a(v2, hw2, dinv, p['b2'], Wpp)
    vsp = _scatter_col0(ut16, src2, dst2, z).reshape(2, NP, 128)
    vs0 = _tk_col0sum(vsp)
    mf, sm, mf16 = _tk3b(vs0, t0, dinv, batch2, p['bp'].reshape(1, 1))
    mpart = _scatter_col0(mf16, src2, dst2, z).reshape(2, NP, 128)
    u3, hw, dinv3 = _tk3c(h2, sm, mf, mpart, p['W3'])
    v3p = _scatter_wide1(u3, src2, dst2, z)
    return _tk3d(v3p, hw, dinv3, mf, p['b3'], batch2)


def kernel(x_a, edge_index_a, batch_a, x_b, edge_index_b, batch_b,
           W1, b1, W2, b2, W3, b3, Wp, bp,
           Wc1, bc1, Wc2, bc2, Wc3, bc3):
    p = {'W1': W1, 'b1': b1, 'W2': W2, 'b2': b2, 'W3': W3, 'b3': b3,
         'Wp': Wp, 'bp': bp}
    z = jnp.zeros((NP, 128), _f32)
    x1 = _gcnnet(x_a, edge_index_a, batch_a, p, z)
    x2 = _gcnnet(x_b, edge_index_b, batch_b, p, z)
    Wc3p = jnp.pad(Wc3, ((0, 0), (0, 127)))
    bc3p = jnp.pad(bc3.reshape(1, 1), ((0, 0), (0, 127)))
    out = _head(x1, x2, Wc1, bc1, Wc2, bc2, Wc3p, bc3p)
    return out[:, 0]


# speedup vs baseline: 14.3918x; 1.0007x over previous
import functools

import jax
import jax.numpy as jnp
from jax import lax
from jax.experimental import pallas as pl
from jax.experimental.pallas import tpu as pltpu
from jax.experimental.pallas import tpu_sc as plsc

N = 10000
NP = 10240
E = 160000
EP = 163840
G = 64
MIN_SCORE = 0.001
NC = 2
NS = 16
ERT = EP // 128
CPT4 = EP // (NS * 128)
CPT1 = EP // (NC * NS * 128)
RPT = NP // NS
EPT_S = EP // (NC * NS)
MB = NP // 1024
BM = 1024

_f32 = jnp.float32
_i32 = jnp.int32


def _sc_mesh():
    return plsc.VectorSubcoreMesh(core_axis_name="c", subcore_axis_name="s",
                                  num_cores=NC, num_subcores=NS)


def _scatter_wide4(u, srcg, dst, z):
    def body(u_hbm, srcg_hbm, dst_hbm, z_hbm, out_hbm,
             sidx, didx, buf0, buf1, acc, sem0, sem1):
        c = lax.axis_index("c")
        s = lax.axis_index("s")
        HC = CPT4 // 2
        pltpu.sync_copy(dst_hbm.at[pl.ds(s * CPT4, CPT4)], didx)
        for r in range(4 // NC):
            cb = NC * r + c
            pltpu.sync_copy(z_hbm.at[pl.ds(s * RPT, RPT)],
                            acc.at[pl.ds(s * RPT, RPT)])
            plsc.subcore_barrier()
            for half in range(2):
                dbase = half * HC
                pltpu.sync_copy(
                    srcg_hbm.at[pl.ds(cb * ERT + s * CPT4 + dbase, HC)],
                    sidx)
                pltpu.async_copy(u_hbm.at[sidx.at[0]], buf0, sem0)

                def step(g, carry, dbase=dbase):
                    j0 = 2 * g
                    pltpu.async_copy(u_hbm.at[sidx.at[j0 + 1]], buf1, sem1)
                    pltpu.make_async_copy(u_hbm.at[sidx.at[j0]], buf0,
                                          sem0).wait()
                    pltpu.sync_copy(buf0, acc.at[didx.at[dbase + j0]],
                                    add=True)

                    @pl.when(j0 + 2 < HC)
                    def _():
                        pltpu.async_copy(u_hbm.at[sidx.at[j0 + 2]], buf0,
                                         sem0)

                    pltpu.make_async_copy(u_hbm.at[sidx.at[j0 + 1]], buf1,
                                          sem1).wait()
                    pltpu.sync_copy(buf1, acc.at[didx.at[dbase + j0 + 1]],
                                    add=True)
                    return carry

                lax.fori_loop(0, HC // 2, step, 0)
            plsc.subcore_barrier()
            pltpu.sync_copy(acc.at[pl.ds(s * RPT, RPT)],
                            out_hbm.at[pl.ds(cb * NP + s * RPT, RPT)])
            plsc.subcore_barrier()

    fn = pl.kernel(
        body,
        out_type=jax.ShapeDtypeStruct((4 * NP, 128), _f32),
        mesh=_sc_mesh(),
        scratch_types=[
            pltpu.VMEM((CPT4 // 2, 128), _i32),
            pltpu.VMEM((CPT4, 128), _i32),
            pltpu.VMEM((128, 128), _f32),
            pltpu.VMEM((128, 128), _f32),
            pltpu.VMEM_SHARED((NP, 128), _f32),
            pltpu.SemaphoreType.DMA,
            pltpu.SemaphoreType.DMA,
        ],
    )
    return fn(u, srcg, dst, z)


def _scatter_wide1(u, src, dst, z, W=128):
    def body(u_hbm, src_hbm, dst_hbm, z_hbm, out_hbm,
             sidx, didx, buf0, buf1, acc, sem0, sem1):
        c = lax.axis_index("c")
        s = lax.axis_index("s")
        base = (c * NS + s) * CPT1
        pltpu.sync_copy(src_hbm.at[pl.ds(base, CPT1)], sidx)
        pltpu.sync_copy(dst_hbm.at[pl.ds(base, CPT1)], didx)
        pltpu.sync_copy(z_hbm.at[pl.ds(s * RPT, RPT)],
                        acc.at[pl.ds(s * RPT, RPT)])
        plsc.subcore_barrier()
        pltpu.async_copy(u_hbm.at[sidx.at[0]], buf0, sem0)

        def step(g, carry):
            j0 = 2 * g
            pltpu.async_copy(u_hbm.at[sidx.at[j0 + 1]], buf1, sem1)
            pltpu.make_async_copy(u_hbm.at[sidx.at[j0]], buf0, sem0).wait()
            pltpu.sync_copy(buf0, acc.at[didx.at[j0]], add=True)

            @pl.when(j0 + 2 < CPT1)
            def _():
                pltpu.async_copy(u_hbm.at[sidx.at[j0 + 2]], buf0, sem0)

            pltpu.make_async_copy(u_hbm.at[sidx.at[j0 + 1]], buf1,
                                  sem1).wait()
            pltpu.sync_copy(buf1, acc.at[didx.at[j0 + 1]], add=True)
            return carry

        lax.fori_loop(0, CPT1 // 2, step, 0)
        plsc.subcore_barrier()
        pltpu.sync_copy(acc.at[pl.ds(s * RPT, RPT)],
                        out_hbm.at[pl.ds(c * NP + s * RPT, RPT)])

    fn = pl.kernel(
        body,
        out_type=jax.ShapeDtypeStruct((NC * NP, W), _f32),
        mesh=_sc_mesh(),
        scratch_types=[
            pltpu.VMEM((CPT1, 128), _i32),
            pltpu.VMEM((CPT1, 128), _i32),
            pltpu.VMEM((128, W), _f32),
            pltpu.VMEM((128, W), _f32),
            pltpu.VMEM_SHARED((NP, W), _f32),
            pltpu.SemaphoreType.DMA,
            pltpu.SemaphoreType.DMA,
        ],
    )
    return fn(u, src, dst, z)


def _scatter_col0(u, src2, dst2, z):
    return _scatter_wide1(u, src2, dst2, z, W=128)


def _elu(x):
    return jnp.where(x > 0, x, jnp.exp(x) - 1.0)


def _tk1(x, W1, degp):
    F = x.shape[1]

    def body(x_r, w_r, dp_r, u1_r, h0_r, dinv_r):
        h0b = jnp.dot(x_r[...], w_r[...], preferred_element_type=_f32)
        dp = dp_r[...]
        deg = (dp[0] + dp[1])[:, :1] + 1.0
        y = lax.rsqrt(deg)
        dinvb = y * (1.5 - 0.5 * deg * y * y)
        u1_r[...] = dinvb * h0b
        h0_r[...] = h0b
        dinv_r[...] = dinvb

    return pl.pallas_call(
        body,
        grid=(MB, 4),
        in_specs=[
            pl.BlockSpec((BM, F), lambda i, j: (i, 0)),
            pl.BlockSpec((F, 128), lambda i, j: (0, j)),
            pl.BlockSpec((2, BM, 128), lambda i, j: (0, i, 0)),
        ],
        out_specs=[
            pl.BlockSpec((BM, 128), lambda i, j: (j * MB + i, 0)),
            pl.BlockSpec((BM, 128), lambda i, j: (i, j)),
            pl.BlockSpec((BM, 1), lambda i, j: (i, 0)),
        ],
        out_shape=[
            jax.ShapeDtypeStruct((4 * NP, 128), _f32),
            jax.ShapeDtypeStruct((NP, 512), _f32),
            jax.ShapeDtypeStruct((NP, 1), _f32),
        ],
    )(x, W1, degp)


def _tk_layer(v, hwprev, dinv, W, b):
    def body(v_r, hp_r, dinv_r, w_r, b_r, u_r, hw_r):
        vv = v_r[...]
        cat = jnp.concatenate([vv[0], vv[1], vv[2], vv[3]], axis=-1)
        d = dinv_r[...]
        h = _elu(d * cat + d * d * hp_r[...] + b_r[...])
        hwb = jnp.dot(h, w_r[...], preferred_element_type=_f32)
        u_r[...] = d * hwb
        hw_r[...] = hwb

    return pl.pallas_call(
        body,
        grid=(MB, 4),
        in_specs=[
            pl.BlockSpec((4, BM, 128), lambda i, j: (0, i, 0)),
            pl.BlockSpec((BM, 512), lambda i, j: (i, 0)),
            pl.BlockSpec((BM, 1), lambda i, j: (i, 0)),
            pl.BlockSpec((512, 128), lambda i, j: (0, j)),
            pl.BlockSpec((1, 512), lambda i, j: (0, 0)),
        ],
        out_specs=[
            pl.BlockSpec((BM, 128), lambda i, j: (j * MB + i, 0)),
            pl.BlockSpec((BM, 128), lambda i, j: (i, j)),
        ],
        out_shape=[
            jax.ShapeDtypeStruct((4 * NP, 128), _f32),
            jax.ShapeDtypeStruct((NP, 512), _f32),
        ],
    )(v.reshape(4, NP, 128), hwprev, dinv, W, b.reshape(1, 512))


def _tk3a(v2, h1, dinv, b2, Wpp):
    def body(v_r, hp_r, dinv_r, b_r, wp_r, h2_r, t0_r, ut_r):
        vv = v_r[...]
        cat = jnp.concatenate([vv[0], vv[1], vv[2], vv[3]], axis=-1)
        d = dinv_r[...]
        h2 = _elu(d * cat + d * d * hp_r[...] + b_r[...])
        tp = jnp.dot(h2, wp_r[...], preferred_element_type=_f32)
        t0 = tp[:, :1]
        h2_r[...] = h2
        t0_r[...] = t0
        col0 = lax.broadcasted_iota(_i32, (BM, 128), 1) == 0
        ut_r[...] = jnp.where(col0, d * t0, 0.0)

    return pl.pallas_call(
        body,
        grid=(MB,),
        in_specs=[
            pl.BlockSpec((4, BM, 128), lambda i: (0, i, 0)),
            pl.BlockSpec((BM, 512), lambda i: (i, 0)),
            pl.BlockSpec((BM, 1), lambda i: (i, 0)),
            pl.BlockSpec((1, 512), lambda i: (0, 0)),
            pl.BlockSpec((512, 128), lambda i: (0, 0)),
        ],
        out_specs=[
            pl.BlockSpec((BM, 512), lambda i: (i, 0)),
            pl.BlockSpec((BM, 1), lambda i: (i, 0)),
            pl.BlockSpec((BM, 128), lambda i: (i, 0)),
        ],
        out_shape=[
            jax.ShapeDtypeStruct((NP, 512), _f32),
            jax.ShapeDtypeStruct((NP, 1), _f32),
            jax.ShapeDtypeStruct((NP, 128), _f32),
        ],
    )(v2.reshape(4, NP, 128), h1, dinv, b2.reshape(1, 512), Wpp)


def _tk_col0sum(vp):
    def body(vp_r, out_r):
        v = vp_r[...]
        out_r[...] = (v[0] + v[1])[:, :1]

    return pl.pallas_call(
        body,
        grid=(MB,),
        in_specs=[pl.BlockSpec((2, BM, 128), lambda i: (0, i, 0))],
        out_specs=pl.BlockSpec((BM, 1), lambda i: (i, 0)),
        out_shape=jax.ShapeDtypeStruct((NP, 1), _f32),
    )(vp)


def _tk3b(vs0, t0, dinv, batch2, bp):
    def body(vs_r, t0_r, dinv_r, b_r, bp_r, mf_r, sm_r, mf16_r):
        vs = vs_r[...]
        d = dinv_r[...]
        s = d * vs + d * d * t0_r[...] + bp_r[0, 0]
        bat = b_r[...]
        iota_g = lax.broadcasted_iota(_i32, (1, G), 1)
        Bb = bat == iota_g
        valid = bat < G
        neg = jnp.float32(-1e30)
        zero = jnp.float32(0.0)
        smax_g = jnp.max(jnp.where(Bb, s, neg), axis=0, keepdims=True)
        smax_n = jnp.sum(jnp.where(Bb, smax_g, zero), axis=1,
                         keepdims=True)
        e = jnp.where(valid, jnp.exp(s - smax_n), 0.0)
        z_g = jnp.sum(jnp.where(Bb, e, zero), axis=0, keepdims=True)
        z_n = jnp.sum(jnp.where(Bb, z_g, zero), axis=1, keepdims=True)
        score = e / (z_n + 1e-16)
        scmax_g = jnp.max(jnp.where(Bb, score, neg), axis=0,
                          keepdims=True)
        scmax_n = jnp.sum(jnp.where(Bb, scmax_g, zero), axis=1,
                          keepdims=True)
        thresh = jnp.minimum(scmax_n - 1e-7, MIN_SCORE)
        sel = jnp.logical_and(valid, score > thresh)
        mfv = jnp.where(sel, 1.0, 0.0)
        mf_r[...] = mfv
        sm_r[...] = jnp.where(sel, score, 0.0)
        col0 = lax.broadcasted_iota(_i32, (NP, 128), 1) == 0
        mf16_r[...] = jnp.where(col0, mfv, 0.0)

    return pl.pallas_call(
        body,
        in_specs=[
            pl.BlockSpec((NP, 1), lambda: (0, 0)),
            pl.BlockSpec((NP, 1), lambda: (0, 0)),
            pl.BlockSpec((NP, 1), lambda: (0, 0)),
            pl.BlockSpec((NP, 1), lambda: (0, 0)),
            pl.BlockSpec((1, 1), lambda: (0, 0)),
        ],
        out_specs=[
            pl.BlockSpec((NP, 1), lambda: (0, 0)),
            pl.BlockSpec((NP, 1), lambda: (0, 0)),
            pl.BlockSpec((NP, 128), lambda: (0, 0)),
        ],
        out_shape=[
            jax.ShapeDtypeStruct((NP, 1), _f32),
            jax.ShapeDtypeStruct((NP, 1), _f32),
            jax.ShapeDtypeStruct((NP, 128), _f32),
        ],
    )(vs0, t0, dinv, batch2, bp)


def _tk3c(h2, sm, mf, mpart, W3):
    def body(h2_r, sm_r, mf_r, mp_r, w_r, u3_r, hw_r, d3_r):
        mp = mp_r[...]
        msum = (mp[0] + mp[1])[:, :1]
        mfv = mf_r[...]
        deg3 = mfv * msum + mfv
        y3 = lax.rsqrt(jnp.maximum(deg3, 1e-30))
        y3 = y3 * (1.5 - 0.5 * deg3 * y3 * y3)
        dinv3 = jnp.where(deg3 > 0, y3, 0.0)
        hx = h2_r[...] * sm_r[...]
        hw = jnp.dot(hx, w_r[...], preferred_element_type=_f32)
        u3_r[...] = dinv3 * hw
        hw_r[...] = hw
        d3_r[...] = dinv3

    return pl.pallas_call(
        body,
        grid=(MB,),
        in_specs=[
            pl.BlockSpec((BM, 512), lambda i: (i, 0)),
            pl.BlockSpec((BM, 1), lambda i: (i, 0)),
            pl.BlockSpec((BM, 1), lambda i: (i, 0)),
            pl.BlockSpec((2, BM, 128), lambda i: (0, i, 0)),
            pl.BlockSpec((512, 128), lambda i: (0, 0)),
        ],
        out_specs=[
            pl.BlockSpec((BM, 128), lambda i: (i, 0)),
            pl.BlockSpec((BM, 128), lambda i: (i, 0)),
            pl.BlockSpec((BM, 1), lambda i: (i, 0)),
        ],
        out_shape=[
            jax.ShapeDtypeStruct((NP, 128), _f32),
            jax.ShapeDtypeStruct((NP, 128), _f32),
            jax.ShapeDtypeStruct((NP, 1), _f32),
        ],
    )(h2, sm, mf, mpart, W3)


def _tk3d(v3p, hw, dinv3, mf, b3, batch2):
    def body(v_r, hw_r, d3_r, mf_r, b_r, bat_r, out_r):
        i = pl.program_id(0)
        vv = v_r[...]
        v3 = vv[0] + vv[1]
        d3 = d3_r[...]
        out3 = d3 * v3 + d3 * d3 * hw_r[...] + b_r[...]
        h3f = _elu(out3) * mf_r[...]
        iota_g = lax.broadcasted_iota(_i32, (1, G), 1)
        B = (bat_r[...] == iota_g).astype(_f32)
        part = lax.dot_general(B, h3f, (((0,), (0,)), ((), ())),
                               preferred_element_type=_f32,
                               precision=lax.Precision.HIGHEST)

        @pl.when(i == 0)
        def _():
            out_r[...] = part

        @pl.when(i > 0)
        def _():
            out_r[...] = out_r[...] + part

    return pl.pallas_call(
        body,
        grid=(MB,),
        in_specs=[
            pl.BlockSpec((2, BM, 128), lambda i: (0, i, 0)),
            pl.BlockSpec((BM, 128), lambda i: (i, 0)),
            pl.BlockSpec((BM, 1), lambda i: (i, 0)),
            pl.BlockSpec((BM, 1), lambda i: (i, 0)),
            pl.BlockSpec((1, 128), lambda i: (0, 0)),
            pl.BlockSpec((BM, 1), lambda i: (i, 0)),
        ],
        out_specs=pl.BlockSpec((G, 128), lambda i: (0, 0)),
        out_shape=jax.ShapeDtypeStruct((G, 128), _f32),
    )(v3p.reshape(2, NP, 128), hw, dinv3, mf, b3.reshape(1, 128), batch2)


def _head(x1, x2, Wc1, bc1, Wc2, bc2, Wc3p, bc3p):
    def body(x1_r, x2_r, w1r, b1r, w2r, b2r, w3r, b3r, outr):
        a, b = x1_r[...], x2_r[...]
        xx = jnp.concatenate([a, b, a - b, a * b], axis=-1)
        hh = jnp.maximum(jnp.dot(xx, w1r[...],
                                 preferred_element_type=_f32) + b1r[...],
                         0.0)
        hh = jnp.maximum(jnp.dot(hh, w2r[...],
                                 preferred_element_type=_f32) + b2r[...],
                         0.0)
        outr[...] = (jnp.dot(hh, w3r[...], preferred_element_type=_f32)
                     + b3r[...])

    return pl.pallas_call(
        body,
        out_shape=jax.ShapeDtypeStruct((G, 128), _f32),
    )(x1, x2, Wc1, bc1.reshape(1, 64), Wc2, bc2.reshape(1, 64),
      Wc3p, bc3p)


def _gcnnet(x, edge_index, batch, p, z):
    src = edge_index[0]
    dst = edge_index[1]
    padi = jnp.arange(EP - E, dtype=_i32)
    srcp = jnp.concatenate([src, padi % 16])
    dstp = jnp.concatenate([dst, N + (padi % (NP - N))])
    src2 = srcp.reshape(ERT, 128)
    dst2 = dstp.reshape(ERT, 128)
    srcg = jnp.concatenate(
        [srcp + cb * NP for cb in range(4)]).reshape(4 * ERT, 128)
    xp = jnp.concatenate(
        [x, jnp.zeros((NP - N, x.shape[1]), _f32)], axis=0)
    batchp = jnp.concatenate(
        [batch.astype(_i32), jnp.full((NP - N,), G, _i32)])
    batch2 = batchp[:, None]

    ones0 = jnp.pad(jnp.ones((NP, 1), _f32), ((0, 0), (0, 127)))

    degp = _scatter_col0(ones0, src2, dst2, z).reshape(2, NP, 128)
    u1, h0, dinv = _tk1(xp, p['W1'], degp)
    v1 = _scatter_wide4(u1, srcg, dst2, z)
    u2, hw2 = _tk_layer(v1, h0, dinv, p['W2'], p['b1'])
    v2 = _scatter_wide4(u2, srcg, dst2, z)
    Wpp = jnp.pad(p['Wp'], ((0, 0), (0, 127)))
    h2, t0, ut16 = _tk3a(v2, hw2, dinv, p['b2'], Wpp)
    vsp = _scatter_col0(ut16, src2, dst2, z).reshape(2, NP, 128)
    vs0 = _tk_col0sum(vsp)
    mf, sm, mf16 = _tk3b(vs0, t0, dinv, batch2, p['bp'].reshape(1, 1))
    mpart = _scatter_col0(mf16, src2, dst2, z).reshape(2, NP, 128)
    u3, hw, dinv3 = _tk3c(h2, sm, mf, mpart, p['W3'])
    v3p = _scatter_wide1(u3, src2, dst2, z)
    return _tk3d(v3p, hw, dinv3, mf, p['b3'], batch2)


def kernel(x_a, edge_index_a, batch_a, x_b, edge_index_b, batch_b,
           W1, b1, W2, b2, W3, b3, Wp, bp,
           Wc1, bc1, Wc2, bc2, Wc3, bc3):
    p = {'W1': W1, 'b1': b1, 'W2': W2, 'b2': b2, 'W3': W3, 'b3': b3,
         'Wp': Wp, 'bp': bp}
    z = jnp.zeros((NP, 128), _f32)
    x1 = _gcnnet(x_a, edge_index_a, batch_a, p, z)
    x2 = _gcnnet(x_b, edge_index_b, batch_b, p, z)
    Wc3p = jnp.pad(Wc3, ((0, 0), (0, 127)))
    bc3p = jnp.pad(bc3.reshape(1, 1), ((0, 0), (0, 127)))
    out = _head(x1, x2, Wc1, bc1, Wc2, bc2, Wc3p, bc3p)
    return out[:, 0]
